# Initial kernel scaffold; baseline (speedup 1.0000x reference)
import jax
import jax.numpy as jnp
from jax.experimental import pallas as pl


def kernel(outcls, label_flatten, instmap):
    raise NotImplementedError("write your pallas kernel here")


# speedup vs baseline: 15.6419x; 15.6419x over previous
import functools

import jax
import jax.numpy as jnp
from jax.experimental import pallas as pl
from jax.experimental.pallas import tpu as pltpu

NUM_INSTANCES = 512
TOPK = 20
ROW_BLOCK = 128
NEG_INF = float("-inf")


def _loss_kernel(out_ref, lab_ref, inst_ref, o_ref, sums, counts, *, nb):
    i = pl.program_id(0)

    @pl.when(i == 0)
    def _init():
        sums[...] = jnp.zeros_like(sums)
        counts[...] = jnp.zeros_like(counts)

    v = out_ref[...]
    lab = lab_ref[...]
    r, c = v.shape
    colid = jax.lax.broadcasted_iota(jnp.int32, (r, c), 1)
    labmask = colid == lab
    tlog = jnp.sum(jnp.where(labmask, v, 0.0), axis=1, keepdims=True)
    work = jnp.where(labmask, -9999.0, v)

    m1 = jnp.max(work, axis=1, keepdims=True)
    mprime = jnp.maximum(m1, tlog)

    s = jnp.exp(tlog - mprime)
    cum = jnp.zeros((r, 1), jnp.float32)
    for t in range(TOPK):
        g = jnp.max(work, axis=1, keepdims=True)
        eqm = work == g
        cnt = jnp.sum(eqm.astype(jnp.float32), axis=1, keepdims=True)
        take = jnp.minimum(cnt, TOPK - cum)
        s = s + take * jnp.exp(g - mprime)
        cum = cum + take
        if t < TOPK - 1:
            work = jnp.where(eqm, NEG_INF, work)

    loss = jnp.log(s) + mprime - tlog
    loss = jnp.where(lab == -1, 0.0, loss)

    inst = inst_ref[...]
    segid = jax.lax.broadcasted_iota(jnp.int32, (r, NUM_INSTANCES), 1)
    onehot = (inst == segid).astype(jnp.float32)
    sums[...] += jnp.sum(onehot * loss, axis=0, keepdims=True)
    counts[...] += jnp.sum(onehot, axis=0, keepdims=True)

    @pl.when(i == nb - 1)
    def _emit():
        o_ref[...] = sums[...] / jnp.maximum(counts[...], 1.0)


@jax.jit
def kernel(outcls, label_flatten, instmap):
    n, c = outcls.shape
    nb = n // ROW_BLOCK
    lab2 = label_flatten.reshape(n, 1)
    inst2 = instmap.reshape(n, 1)
    out = pl.pallas_call(
        functools.partial(_loss_kernel, nb=nb),
        grid=(nb,),
        in_specs=[
            pl.BlockSpec((ROW_BLOCK, c), lambda i: (i, 0)),
            pl.BlockSpec((ROW_BLOCK, 1), lambda i: (i, 0)),
            pl.BlockSpec((ROW_BLOCK, 1), lambda i: (i, 0)),
        ],
        out_specs=pl.BlockSpec((1, NUM_INSTANCES), lambda i: (0, 0)),
        out_shape=jax.ShapeDtypeStruct((1, NUM_INSTANCES), jnp.float32),
        scratch_shapes=[
            pltpu.VMEM((1, NUM_INSTANCES), jnp.float32),
            pltpu.VMEM((1, NUM_INSTANCES), jnp.float32),
        ],
        compiler_params=pltpu.CompilerParams(
            dimension_semantics=("arbitrary",),
        ),
    )(outcls, lab2, inst2)
    return out.reshape(NUM_INSTANCES)
